# baseline (speedup 1.0000x reference)
import functools

import jax
import jax.numpy as jnp
from jax import lax
from jax.experimental import pallas as pl
from jax.experimental.pallas import tpu as pltpu
from jax.experimental.pallas import tpu_sc as plsc

_N_NEURON = 8192
_KV = 2048
_KP = 1024
_K = _KV + _KP
_B = 4096

_MBLK = 512

_NC = 2
_NS = 16
_BS = 128
_NSTRIP = _B // _BS
_NNC = _N_NEURON // _NC
_IDS_PER_TEC = _K // _NS
_IDC = 96
_NROW = _NNC // _NS
_IGNORED = -1


def _embed_t_body(vis_ref, prp_ref, wv_ref, bv_ref, wp_ref, bp_ref, src_ref):
    sv = lax.dot_general(
        wv_ref[...], vis_ref[...], (((0,), (1,)), ((), ())),
        preferred_element_type=jnp.float32,
    )
    src_ref[: _KV, :] = jnp.maximum(sv + bv_ref[...], 0.0)
    sp = lax.dot_general(
        wp_ref[...], prp_ref[...], (((0,), (1,)), ((), ())),
        preferred_element_type=jnp.float32,
    )
    src_ref[_KV:, :] = jnp.maximum(sp + bp_ref[...], 0.0)


def _embed_t(vision, proprio, W_vision, b_vision, W_proprio, b_proprio):
    grid = (_B // _MBLK,)
    return pl.pallas_call(
        _embed_t_body,
        grid=grid,
        in_specs=[
            pl.BlockSpec((_MBLK, 1024), lambda i: (i, 0)),
            pl.BlockSpec((_MBLK, 512), lambda i: (i, 0)),
            pl.BlockSpec((1024, _KV), lambda i: (0, 0)),
            pl.BlockSpec((_KV, 1), lambda i: (0, 0)),
            pl.BlockSpec((512, _KP), lambda i: (0, 0)),
            pl.BlockSpec((_KP, 1), lambda i: (0, 0)),
        ],
        out_specs=pl.BlockSpec((_K, _MBLK), lambda i: (0, i)),
        out_shape=jax.ShapeDtypeStruct((_K, _B), jnp.float32),
    )(vision, proprio, W_vision, b_vision.reshape(_KV, 1),
      W_proprio, b_proprio.reshape(_KP, 1))


@functools.partial(
    pl.kernel,
    out_type=jax.ShapeDtypeStruct((_N_NEURON, _B), jnp.float32),
    mesh=plsc.VectorSubcoreMesh(core_axis_name="c", subcore_axis_name="s"),
    scratch_types=[
        pltpu.VMEM((_IDS_PER_TEC,), jnp.int32),
        pltpu.VMEM((_IDS_PER_TEC // _IDC, _IDC), jnp.int32),
        pltpu.VMEM((_IDS_PER_TEC, _BS), jnp.float32),
        pltpu.VMEM((_NROW, _BS), jnp.float32),
        pltpu.VMEM_SHARED((_NNC, _BS), jnp.float32),
    ],
)
def _sc_scatter_t(src_hbm, ids_hbm, out_hbm, raw_v, ids_v, src_v, zero_v,
                  acc):
    s = lax.axis_index("s")
    c = lax.axis_index("c")

    pltpu.sync_copy(ids_hbm.at[pl.ds(s * _IDS_PER_TEC, _IDS_PER_TEC)], raw_v)

    def adjloop(i, _):
        q = i // (_IDC // 16)
        j = i % (_IDC // 16)
        raw = raw_v[pl.ds(q * _IDC + j * 16, 16)]
        local = raw - c * _NNC
        ok = (local >= 0) & (local < _NNC)
        ids_v[q, pl.ds(j * 16, 16)] = jnp.where(ok, local, _IGNORED)
        return 0

    lax.fori_loop(0, _IDS_PER_TEC // 16, adjloop, 0, unroll=True)

    zeros16 = jnp.zeros((16,), jnp.float32)

    def zloop(i, _):
        r = i // (_BS // 16)
        q = i % (_BS // 16)
        zero_v[r, pl.ds(q * 16, 16)] = zeros16
        return 0

    lax.fori_loop(0, _NROW * _BS // 16, zloop, 0, unroll=8)

    nrow0 = s * _NROW

    def sloop(m, _):
        c0 = m * _BS
        pltpu.sync_copy(zero_v, acc.at[pl.ds(nrow0, _NROW)])
        pltpu.sync_copy(
            src_hbm.at[pl.ds(s * _IDS_PER_TEC, _IDS_PER_TEC),
                       pl.ds(c0, _BS)],
            src_v,
        )
        plsc.subcore_barrier()

        def scloop(q, _):
            pltpu.sync_copy(
                src_v.at[pl.ds(q * _IDC, _IDC)],
                acc.at[plsc.Indices(ids_v.at[q], ignored_value=_IGNORED)],
                add=True,
            )
            return 0

        lax.fori_loop(0, _IDS_PER_TEC // _IDC, scloop, 0, unroll=True)
        plsc.subcore_barrier()
        pltpu.sync_copy(
            acc.at[pl.ds(nrow0, _NROW)],
            out_hbm.at[pl.ds(c * _NNC + nrow0, _NROW), pl.ds(c0, _BS)],
        )
        plsc.subcore_barrier()
        return 0

    lax.fori_loop(0, _NSTRIP, sloop, 0)


def _transpose_body(in_ref, out_ref):
    out_ref[...] = jnp.swapaxes(in_ref[...], 0, 1)


def _transpose(outT):
    grid = (_B // 512, _N_NEURON // 512)
    return pl.pallas_call(
        _transpose_body,
        grid=grid,
        in_specs=[pl.BlockSpec((512, 512), lambda i, j: (j, i))],
        out_specs=pl.BlockSpec((512, 512), lambda i, j: (i, j)),
        out_shape=jax.ShapeDtypeStruct((_B, _N_NEURON), jnp.float32),
    )(outT)


def kernel(vision, proprio, W_vision, b_vision, W_proprio, b_proprio,
           ids_vision, ids_proprio):
    srcT = _embed_t(vision, proprio, W_vision, b_vision, W_proprio, b_proprio)
    ids = jnp.concatenate([ids_vision, ids_proprio])
    outT = _sc_scatter_t(srcT, ids)
    return _transpose(outT)


# speedup vs baseline: 1.0957x; 1.0957x over previous
import functools

import jax
import jax.numpy as jnp
from jax import lax
from jax.experimental import pallas as pl
from jax.experimental.pallas import tpu as pltpu
from jax.experimental.pallas import tpu_sc as plsc

_N_NEURON = 8192
_KV = 2048
_KP = 1024
_K = _KV + _KP
_B = 4096

_BSC = 2560
_BTC = _B - _BSC

_MBLK = 512
_KB = 512

_NC = 2
_NS = 16
_BS = 128
_NSTRIP = _BSC // _BS
_NNC = _N_NEURON // _NC
_IDS_PER_TEC = _K // _NS
_IDC = 96
_NROW = _NNC // _NS
_IGNORED = -1


def _embed_t_body(vis_ref, prp_ref, wv_ref, bv_ref, wp_ref, bp_ref, src_ref):
    sv = lax.dot_general(
        wv_ref[...], vis_ref[...], (((0,), (1,)), ((), ())),
        preferred_element_type=jnp.float32,
    )
    src_ref[: _KV, :] = jnp.maximum(sv + bv_ref[...], 0.0)
    sp = lax.dot_general(
        wp_ref[...], prp_ref[...], (((0,), (1,)), ((), ())),
        preferred_element_type=jnp.float32,
    )
    src_ref[_KV:, :] = jnp.maximum(sp + bp_ref[...], 0.0)


def _embed_t(vision, proprio, W_vision, b_vision, W_proprio, b_proprio):
    grid = (_BSC // _MBLK,)
    return pl.pallas_call(
        _embed_t_body,
        grid=grid,
        in_specs=[
            pl.BlockSpec((_MBLK, 1024), lambda i: (i, 0)),
            pl.BlockSpec((_MBLK, 512), lambda i: (i, 0)),
            pl.BlockSpec((1024, _KV), lambda i: (0, 0)),
            pl.BlockSpec((_KV, 1), lambda i: (0, 0)),
            pl.BlockSpec((512, _KP), lambda i: (0, 0)),
            pl.BlockSpec((_KP, 1), lambda i: (0, 0)),
        ],
        out_specs=pl.BlockSpec((_K, _MBLK), lambda i: (0, i)),
        out_shape=jax.ShapeDtypeStruct((_K, _BSC), jnp.float32),
    )(vision, proprio, W_vision, b_vision.reshape(_KV, 1),
      W_proprio, b_proprio.reshape(_KP, 1))


@functools.partial(
    pl.kernel,
    out_type=jax.ShapeDtypeStruct((_N_NEURON, _BSC), jnp.float32),
    mesh=plsc.VectorSubcoreMesh(core_axis_name="c", subcore_axis_name="s"),
    scratch_types=[
        pltpu.VMEM((_IDS_PER_TEC,), jnp.int32),
        pltpu.VMEM((_IDS_PER_TEC // _IDC, _IDC), jnp.int32),
        pltpu.VMEM((_IDS_PER_TEC, _BS), jnp.float32),
        pltpu.VMEM((_NROW, _BS), jnp.float32),
        pltpu.VMEM_SHARED((_NNC, _BS), jnp.float32),
    ],
)
def _sc_scatter_t(src_hbm, ids_hbm, out_hbm, raw_v, ids_v, src_v, zero_v,
                  acc):
    s = lax.axis_index("s")
    c = lax.axis_index("c")

    pltpu.sync_copy(ids_hbm.at[pl.ds(s * _IDS_PER_TEC, _IDS_PER_TEC)], raw_v)

    def adjloop(i, _):
        q = i // (_IDC // 16)
        j = i % (_IDC // 16)
        raw = raw_v[pl.ds(q * _IDC + j * 16, 16)]
        local = raw - c * _NNC
        ok = (local >= 0) & (local < _NNC)
        ids_v[q, pl.ds(j * 16, 16)] = jnp.where(ok, local, _IGNORED)
        return 0

    lax.fori_loop(0, _IDS_PER_TEC // 16, adjloop, 0, unroll=True)

    zeros16 = jnp.zeros((16,), jnp.float32)

    def zloop(i, _):
        r = i // (_BS // 16)
        q = i % (_BS // 16)
        zero_v[r, pl.ds(q * 16, 16)] = zeros16
        return 0

    lax.fori_loop(0, _NROW * _BS // 16, zloop, 0, unroll=8)

    nrow0 = s * _NROW

    def sloop(m, _):
        c0 = m * _BS
        pltpu.sync_copy(zero_v, acc.at[pl.ds(nrow0, _NROW)])
        pltpu.sync_copy(
            src_hbm.at[pl.ds(s * _IDS_PER_TEC, _IDS_PER_TEC),
                       pl.ds(c0, _BS)],
            src_v,
        )
        plsc.subcore_barrier()

        def scloop(q, _):
            pltpu.sync_copy(
                src_v.at[pl.ds(q * _IDC, _IDC)],
                acc.at[plsc.Indices(ids_v.at[q], ignored_value=_IGNORED)],
                add=True,
            )
            return 0

        lax.fori_loop(0, _IDS_PER_TEC // _IDC, scloop, 0, unroll=True)
        plsc.subcore_barrier()
        pltpu.sync_copy(
            acc.at[pl.ds(nrow0, _NROW)],
            out_hbm.at[pl.ds(c * _NNC + nrow0, _NROW), pl.ds(c0, _BS)],
        )
        plsc.subcore_barrier()
        return 0

    lax.fori_loop(0, _NSTRIP, sloop, 0)


def _transpose_body(in_ref, out_ref):
    out_ref[...] = jnp.swapaxes(in_ref[...], 0, 1)


def _transpose(outT):
    grid = (_BSC // 512, _N_NEURON // 512)
    return pl.pallas_call(
        _transpose_body,
        grid=grid,
        in_specs=[pl.BlockSpec((512, 512), lambda i, j: (j, i))],
        out_specs=pl.BlockSpec((512, 512), lambda i, j: (i, j)),
        out_shape=jax.ShapeDtypeStruct((_BSC, _N_NEURON), jnp.float32),
    )(outT)


def _onehot_body(ids_ref, vis_ref, prp_ref, wv_ref, bv_ref, wp_ref, bp_ref,
                 out_ref, src_ref):
    sv = jnp.dot(vis_ref[...], wv_ref[...], preferred_element_type=jnp.float32)
    sv = jnp.maximum(sv + bv_ref[...], 0.0)
    sp = jnp.dot(prp_ref[...], wp_ref[...], preferred_element_type=jnp.float32)
    sp = jnp.maximum(sp + bp_ref[...], 0.0)
    src_ref[:, :_KV] = sv.astype(jnp.bfloat16)
    src_ref[:, _KV:] = sp.astype(jnp.bfloat16)

    out_ref[...] = jnp.zeros_like(out_ref)

    def kb_step(i, _):
        idb = ids_ref[pl.ds(i * _KB, _KB), :]
        cols = jax.lax.broadcasted_iota(jnp.int32, (_KB, _N_NEURON), 1)
        onehot = (idb == cols).astype(jnp.bfloat16)
        sb = src_ref[:, pl.ds(i * _KB, _KB)]
        out_ref[...] += jnp.dot(sb, onehot, preferred_element_type=jnp.float32)
        return 0

    jax.lax.fori_loop(0, _K // _KB, kb_step, 0)


def _onehot_scatter(ids, vision, proprio, W_vision, b_vision, W_proprio,
                    b_proprio):
    grid = (_BTC // _MBLK,)
    return pl.pallas_call(
        _onehot_body,
        grid=grid,
        in_specs=[
            pl.BlockSpec((_K, 1), lambda i: (0, 0)),
            pl.BlockSpec((_MBLK, 1024), lambda i: (i, 0)),
            pl.BlockSpec((_MBLK, 512), lambda i: (i, 0)),
            pl.BlockSpec((1024, _KV), lambda i: (0, 0)),
            pl.BlockSpec((1, _KV), lambda i: (0, 0)),
            pl.BlockSpec((512, _KP), lambda i: (0, 0)),
            pl.BlockSpec((1, _KP), lambda i: (0, 0)),
        ],
        out_specs=pl.BlockSpec((_MBLK, _N_NEURON), lambda i: (i, 0)),
        out_shape=jax.ShapeDtypeStruct((_BTC, _N_NEURON), jnp.float32),
        scratch_shapes=[pltpu.VMEM((_MBLK, _K), jnp.bfloat16)],
    )(ids, vision, proprio, W_vision, b_vision.reshape(1, _KV),
      W_proprio, b_proprio.reshape(1, _KP))


def kernel(vision, proprio, W_vision, b_vision, W_proprio, b_proprio,
           ids_vision, ids_proprio):
    ids = jnp.concatenate([ids_vision, ids_proprio])

    srcT = _embed_t(vision[:_BSC], proprio[:_BSC], W_vision, b_vision,
                    W_proprio, b_proprio)
    outT = _sc_scatter_t(srcT, ids)

    out_tc = _onehot_scatter(ids.reshape(_K, 1), vision[_BSC:],
                             proprio[_BSC:], W_vision, b_vision, W_proprio,
                             b_proprio)

    out_sc = _transpose(outT)
    return jnp.concatenate([out_sc, out_tc], axis=0)


# speedup vs baseline: 1.1971x; 1.0925x over previous
import functools

import jax
import jax.numpy as jnp
from jax import lax
from jax.experimental import pallas as pl
from jax.experimental.pallas import tpu as pltpu
from jax.experimental.pallas import tpu_sc as plsc

_N_NEURON = 8192
_KV = 2048
_KP = 1024
_K = _KV + _KP
_B = 4096

_BSC = 2048
_BTC = _B - _BSC

_MBLK = 512
_KB = 512

_NC = 2
_NS = 16
_BS = 128
_NSTRIP = _BSC // _BS
_NNC = _N_NEURON // _NC
_IDS_PER_TEC = _K // _NS
_IDC = 96
_NROW = _NNC // _NS
_IGNORED = -1


def _embed_t_body(vis_ref, prp_ref, wv_ref, bv_ref, wp_ref, bp_ref, src_ref):
    sv = lax.dot_general(
        wv_ref[...], vis_ref[...], (((0,), (1,)), ((), ())),
        preferred_element_type=jnp.float32,
    )
    src_ref[: _KV, :] = jnp.maximum(sv + bv_ref[...], 0.0)
    sp = lax.dot_general(
        wp_ref[...], prp_ref[...], (((0,), (1,)), ((), ())),
        preferred_element_type=jnp.float32,
    )
    src_ref[_KV:, :] = jnp.maximum(sp + bp_ref[...], 0.0)


def _embed_t(vision, proprio, W_vision, b_vision, W_proprio, b_proprio):
    grid = (_BSC // _MBLK,)
    return pl.pallas_call(
        _embed_t_body,
        grid=grid,
        in_specs=[
            pl.BlockSpec((_MBLK, 1024), lambda i: (i, 0)),
            pl.BlockSpec((_MBLK, 512), lambda i: (i, 0)),
            pl.BlockSpec((1024, _KV), lambda i: (0, 0)),
            pl.BlockSpec((_KV, 1), lambda i: (0, 0)),
            pl.BlockSpec((512, _KP), lambda i: (0, 0)),
            pl.BlockSpec((_KP, 1), lambda i: (0, 0)),
        ],
        out_specs=pl.BlockSpec((_K, _MBLK), lambda i: (0, i)),
        out_shape=jax.ShapeDtypeStruct((_K, _BSC), jnp.float32),
    )(vision, proprio, W_vision, b_vision.reshape(_KV, 1),
      W_proprio, b_proprio.reshape(_KP, 1))


@functools.partial(
    pl.kernel,
    out_type=jax.ShapeDtypeStruct((_N_NEURON, _BSC), jnp.float32),
    mesh=plsc.VectorSubcoreMesh(core_axis_name="c", subcore_axis_name="s"),
    cost_estimate=pl.CostEstimate(
        flops=0, bytes_accessed=400_000_000, transcendentals=0
    ),
    scratch_types=[
        pltpu.VMEM((_IDS_PER_TEC,), jnp.int32),
        pltpu.VMEM((_IDS_PER_TEC // _IDC, _IDC), jnp.int32),
        pltpu.VMEM((_IDS_PER_TEC, _BS), jnp.float32),
        pltpu.VMEM((_NROW, _BS), jnp.float32),
        pltpu.VMEM_SHARED((_NNC, _BS), jnp.float32),
    ],
)
def _sc_scatter_t(src_hbm, ids_hbm, out_hbm, raw_v, ids_v, src_v, zero_v,
                  acc):
    s = lax.axis_index("s")
    c = lax.axis_index("c")

    pltpu.sync_copy(ids_hbm.at[pl.ds(s * _IDS_PER_TEC, _IDS_PER_TEC)], raw_v)

    def adjloop(i, _):
        q = i // (_IDC // 16)
        j = i % (_IDC // 16)
        raw = raw_v[pl.ds(q * _IDC + j * 16, 16)]
        local = raw - c * _NNC
        ok = (local >= 0) & (local < _NNC)
        ids_v[q, pl.ds(j * 16, 16)] = jnp.where(ok, local, _IGNORED)
        return 0

    lax.fori_loop(0, _IDS_PER_TEC // 16, adjloop, 0, unroll=True)

    zeros16 = jnp.zeros((16,), jnp.float32)

    def zloop(i, _):
        r = i // (_BS // 16)
        q = i % (_BS // 16)
        zero_v[r, pl.ds(q * 16, 16)] = zeros16
        return 0

    lax.fori_loop(0, _NROW * _BS // 16, zloop, 0, unroll=8)

    nrow0 = s * _NROW

    def sloop(m, _):
        c0 = m * _BS
        pltpu.sync_copy(zero_v, acc.at[pl.ds(nrow0, _NROW)])
        pltpu.sync_copy(
            src_hbm.at[pl.ds(s * _IDS_PER_TEC, _IDS_PER_TEC),
                       pl.ds(c0, _BS)],
            src_v,
        )
        plsc.subcore_barrier()

        def scloop(q, _):
            pltpu.sync_copy(
                src_v.at[pl.ds(q * _IDC, _IDC)],
                acc.at[plsc.Indices(ids_v.at[q], ignored_value=_IGNORED)],
                add=True,
            )
            return 0

        lax.fori_loop(0, _IDS_PER_TEC // _IDC, scloop, 0, unroll=True)
        plsc.subcore_barrier()
        pltpu.sync_copy(
            acc.at[pl.ds(nrow0, _NROW)],
            out_hbm.at[pl.ds(c * _NNC + nrow0, _NROW), pl.ds(c0, _BS)],
        )
        plsc.subcore_barrier()
        return 0

    lax.fori_loop(0, _NSTRIP, sloop, 0)


_NBI = _BSC // 512


def _merge_body(outT_ref, tc_ref, out_ref):
    i = pl.program_id(0)

    @pl.when(i < _NBI)
    def _():
        out_ref[...] = jnp.swapaxes(outT_ref[...], 0, 1)

    @pl.when(i >= _NBI)
    def _():
        out_ref[...] = tc_ref[...]


def _merge(outT, out_tc):
    grid = (_B // 512, _N_NEURON // 512)
    return pl.pallas_call(
        _merge_body,
        grid=grid,
        in_specs=[
            pl.BlockSpec((512, 512),
                         lambda i, j: (j, jnp.minimum(i, _NBI - 1))),
            pl.BlockSpec((512, 512),
                         lambda i, j: (jnp.maximum(i - _NBI, 0), j)),
        ],
        out_specs=pl.BlockSpec((512, 512), lambda i, j: (i, j)),
        out_shape=jax.ShapeDtypeStruct((_B, _N_NEURON), jnp.float32),
    )(outT, out_tc)


def _onehot_body(ids_ref, vis_ref, prp_ref, wv_ref, bv_ref, wp_ref, bp_ref,
                 out_ref, src_ref):
    sv = jnp.dot(vis_ref[...], wv_ref[...], preferred_element_type=jnp.float32)
    sv = jnp.maximum(sv + bv_ref[...], 0.0)
    sp = jnp.dot(prp_ref[...], wp_ref[...], preferred_element_type=jnp.float32)
    sp = jnp.maximum(sp + bp_ref[...], 0.0)
    src_ref[:, :_KV] = sv.astype(jnp.bfloat16)
    src_ref[:, _KV:] = sp.astype(jnp.bfloat16)

    out_ref[...] = jnp.zeros_like(out_ref)

    def kb_step(i, _):
        idb = ids_ref[pl.ds(i * _KB, _KB), :]
        cols = jax.lax.broadcasted_iota(jnp.int32, (_KB, _N_NEURON), 1)
        onehot = (idb == cols).astype(jnp.bfloat16)
        sb = src_ref[:, pl.ds(i * _KB, _KB)]
        out_ref[...] += jnp.dot(sb, onehot, preferred_element_type=jnp.float32)
        return 0

    jax.lax.fori_loop(0, _K // _KB, kb_step, 0)


def _onehot_scatter(ids, vision, proprio, W_vision, b_vision, W_proprio,
                    b_proprio):
    grid = (_BTC // _MBLK,)
    return pl.pallas_call(
        _onehot_body,
        grid=grid,
        in_specs=[
            pl.BlockSpec((_K, 1), lambda i: (0, 0)),
            pl.BlockSpec((_MBLK, 1024), lambda i: (i, 0)),
            pl.BlockSpec((_MBLK, 512), lambda i: (i, 0)),
            pl.BlockSpec((1024, _KV), lambda i: (0, 0)),
            pl.BlockSpec((1, _KV), lambda i: (0, 0)),
            pl.BlockSpec((512, _KP), lambda i: (0, 0)),
            pl.BlockSpec((1, _KP), lambda i: (0, 0)),
        ],
        out_specs=pl.BlockSpec((_MBLK, _N_NEURON), lambda i: (i, 0)),
        out_shape=jax.ShapeDtypeStruct((_BTC, _N_NEURON), jnp.float32),
        scratch_shapes=[pltpu.VMEM((_MBLK, _K), jnp.bfloat16)],
    )(ids, vision, proprio, W_vision, b_vision.reshape(1, _KV),
      W_proprio, b_proprio.reshape(1, _KP))


def kernel(vision, proprio, W_vision, b_vision, W_proprio, b_proprio,
           ids_vision, ids_proprio):
    ids = jnp.concatenate([ids_vision, ids_proprio])

    srcT = _embed_t(vision[:_BSC], proprio[:_BSC], W_vision, b_vision,
                    W_proprio, b_proprio)
    outT = _sc_scatter_t(srcT, ids)

    out_tc = _onehot_scatter(ids.reshape(_K, 1), vision[_BSC:],
                             proprio[_BSC:], W_vision, b_vision, W_proprio,
                             b_proprio)

    return _merge(outT, out_tc)
